# baseline (speedup 1.0000x reference)
import functools

import jax
import jax.numpy as jnp
from jax import lax
from jax.experimental import pallas as pl
from jax.experimental.pallas import tpu as pltpu
from jax.experimental.pallas import tpu_sc as plsc

N = 10000
E = 320000
D = 128
H = 128
C = 64
G = 128

NC = 2
NS = 16
NW = NC * NS

CH = 128
N_CHUNKS = 80
W_EDGES = N_CHUNKS * CH
EP = NW * W_EDGES
NP = 10240
ROWS_PER_SUB = NP // NS
OUT_A = 624
OUT_TAIL = N - NS * OUT_A


def _copy_out(acc_sh, out_hbm, cid, sid):
  pltpu.sync_copy(
      acc_sh.at[pl.ds(sid * OUT_A, OUT_A)],
      out_hbm.at[cid].at[pl.ds(sid * OUT_A, OUT_A)],
  )

  @pl.when(sid == 0)
  def _():
    pltpu.sync_copy(
        acc_sh.at[pl.ds(NS * OUT_A, OUT_TAIL)],
        out_hbm.at[cid].at[pl.ds(NS * OUT_A, OUT_TAIL)],
    )


@functools.cache
def _make_sc_segment_sum():
  mesh = plsc.VectorSubcoreMesh(core_axis_name="c", subcore_axis_name="s")

  @functools.partial(
      pl.kernel,
      out_type=jax.ShapeDtypeStruct((NC, N, H), jnp.float32),
      mesh=mesh,
      scratch_types=[
          pltpu.VMEM((CH,), jnp.int32),
          pltpu.VMEM((CH,), jnp.int32),
          pltpu.VMEM((CH,), jnp.int32),
          pltpu.VMEM((CH,), jnp.int32),
          pltpu.VMEM((CH, H), jnp.float32),
          pltpu.VMEM((CH, H), jnp.float32),
          pltpu.VMEM_SHARED((NP, H), jnp.float32),
          pltpu.SemaphoreType.DMA,
          pltpu.SemaphoreType.DMA,
      ],
  )
  def sc_segment_sum(h_hbm, src_hbm, dst_hbm, zeros_hbm, out_hbm,
                     src_a, src_b, dst_a, dst_b, rows_a, rows_b, acc_sh,
                     sem_a, sem_b):
    cid = lax.axis_index("c")
    sid = lax.axis_index("s")

    @pl.loop(0, ROWS_PER_SUB // CH)
    def _(k):
      pltpu.sync_copy(zeros_hbm,
                      acc_sh.at[pl.ds(sid * ROWS_PER_SUB + k * CH, CH)])

    wid = sid * NC + cid
    base = wid * W_EDGES

    def load_idx(c, src_v, dst_v):
      pltpu.sync_copy(src_hbm.at[pl.ds(base + c * CH, CH)], src_v)
      pltpu.sync_copy(dst_hbm.at[pl.ds(base + c * CH, CH)], dst_v)

    def gather(src_v, rows, sem):
      pltpu.async_copy(h_hbm.at[src_v], rows, sem)

    def wait_gather(src_v, rows, sem):
      pltpu.make_async_copy(h_hbm.at[src_v], rows, sem).wait()

    def scatter(rows, dst_v):
      pltpu.sync_copy(rows, acc_sh.at[dst_v], add=True)

    plsc.subcore_barrier()

    load_idx(0, src_a, dst_a)
    gather(src_a, rows_a, sem_a)

    @pl.loop(0, N_CHUNKS - 2, step=2)
    def _(t):
      load_idx(t + 1, src_b, dst_b)
      wait_gather(src_a, rows_a, sem_a)
      gather(src_b, rows_b, sem_b)
      scatter(rows_a, dst_a)
      load_idx(t + 2, src_a, dst_a)
      wait_gather(src_b, rows_b, sem_b)
      gather(src_a, rows_a, sem_a)
      scatter(rows_b, dst_b)

    load_idx(N_CHUNKS - 1, src_b, dst_b)
    wait_gather(src_a, rows_a, sem_a)
    gather(src_b, rows_b, sem_b)
    scatter(rows_a, dst_a)
    wait_gather(src_b, rows_b, sem_b)
    scatter(rows_b, dst_b)

    plsc.subcore_barrier()
    _copy_out(acc_sh, out_hbm, cid, sid)

  return sc_segment_sum


@functools.cache
def _make_sc_degree():
  mesh = plsc.VectorSubcoreMesh(core_axis_name="c", subcore_axis_name="s")

  @functools.partial(
      pl.kernel,
      out_type=jax.ShapeDtypeStruct((NC, N, H), jnp.float32),
      mesh=mesh,
      scratch_types=[
          pltpu.VMEM((CH,), jnp.int32),
          pltpu.VMEM((CH, H), jnp.float32),
          pltpu.VMEM_SHARED((NP, H), jnp.float32),
      ],
  )
  def sc_degree(dst_hbm, ones_hbm, zeros_hbm, out_hbm,
                dst_v, ones_v, acc_sh):
    cid = lax.axis_index("c")
    sid = lax.axis_index("s")

    pltpu.sync_copy(ones_hbm, ones_v)

    @pl.loop(0, ROWS_PER_SUB // CH)
    def _(k):
      pltpu.sync_copy(zeros_hbm,
                      acc_sh.at[pl.ds(sid * ROWS_PER_SUB + k * CH, CH)])

    wid = sid * NC + cid

    plsc.subcore_barrier()

    @pl.loop(0, N_CHUNKS)
    def _(c):
      pltpu.sync_copy(dst_hbm.at[pl.ds(wid * W_EDGES + c * CH, CH)], dst_v)
      pltpu.sync_copy(ones_v, acc_sh.at[dst_v], add=True)

    plsc.subcore_barrier()
    _copy_out(acc_sh, out_hbm, cid, sid)

  return sc_degree


_RB = 400
_NB = N // _RB


def _combine_body(m_ref, c_ref, h_ref, wl_ref, wr_ref, b_ref, o_ref):
  cnt = c_ref[0][:, 0:1] + c_ref[1][:, 0:1]
  inv = 1.0 / jnp.maximum(cnt, 1.0)
  agg = (m_ref[0] + m_ref[1]) * inv
  z = (jnp.dot(agg, wl_ref[...], preferred_element_type=jnp.float32)
       + jnp.dot(h_ref[...], wr_ref[...], preferred_element_type=jnp.float32)
       + b_ref[...])
  o_ref[...] = jnp.maximum(z, 0.0)


def _tc_combine(msg, cntp, h, wl, wr, b):
  return pl.pallas_call(
      _combine_body,
      grid=(_NB,),
      in_specs=[
          pl.BlockSpec((NC, _RB, H), lambda i: (0, i, 0)),
          pl.BlockSpec((NC, _RB, H), lambda i: (0, i, 0)),
          pl.BlockSpec((_RB, H), lambda i: (i, 0)),
          pl.BlockSpec((H, H), lambda i: (0, 0)),
          pl.BlockSpec((H, H), lambda i: (0, 0)),
          pl.BlockSpec((1, H), lambda i: (0, 0)),
      ],
      out_specs=pl.BlockSpec((_RB, H), lambda i: (i, 0)),
      out_shape=jax.ShapeDtypeStruct((N, H), jnp.float32),
  )(msg, cntp, h, wl, wr, b.reshape(1, H))


def _pool_body(h_ref, b_ref, wo_ref, bo_ref, o_ref, acc_ref, cacc_ref):
  i = pl.program_id(0)

  @pl.when(i == 0)
  def _():
    acc_ref[...] = jnp.zeros_like(acc_ref)
    cacc_ref[...] = jnp.zeros_like(cacc_ref)

  h = h_ref[...]
  bidx = b_ref[...]
  iota_g = lax.broadcasted_iota(jnp.int32, (_RB, G), 1)
  onehot = (bidx == iota_g).astype(jnp.float32)
  acc_ref[...] += lax.dot_general(
      onehot, h, (((0,), (0,)), ((), ())), preferred_element_type=jnp.float32)
  cacc_ref[...] += lax.dot_general(
      onehot, jnp.ones((_RB, G), jnp.float32), (((0,), (0,)), ((), ())),
      preferred_element_type=jnp.float32)

  @pl.when(i == _NB - 1)
  def _():
    cnt = jnp.maximum(cacc_ref[:, 0:1], 1.0)
    pooled = acc_ref[...] / cnt
    logits = (jnp.dot(pooled, wo_ref[...], preferred_element_type=jnp.float32)
              + bo_ref[...])
    m = jnp.max(logits, axis=1, keepdims=True)
    lse = jnp.log(jnp.sum(jnp.exp(logits - m), axis=1, keepdims=True)) + m
    o_ref[...] = logits - lse


def _tc_pool(h, batch2d, wo, bo):
  return pl.pallas_call(
      _pool_body,
      grid=(_NB,),
      in_specs=[
          pl.BlockSpec((_RB, H), lambda i: (i, 0)),
          pl.BlockSpec((_RB, 1), lambda i: (i, 0)),
          pl.BlockSpec((H, C), lambda i: (0, 0)),
          pl.BlockSpec((1, C), lambda i: (0, 0)),
      ],
      out_specs=pl.BlockSpec((G, C), lambda i: (0, 0)),
      out_shape=jax.ShapeDtypeStruct((G, C), jnp.float32),
      scratch_shapes=[
          pltpu.VMEM((G, H), jnp.float32),
          pltpu.VMEM((G, G), jnp.float32),
      ],
  )(h, batch2d, wo, bo.reshape(1, C))


def kernel(x, edge_index, batch, Wl1, Wr1, b1, Wl2, Wr2, b2, Wl3, Wr3, b3,
           Wl4, Wr4, b4, Wl5, Wr5, b5, Wo, bo):
  src = edge_index[0].astype(jnp.int32)
  dst = edge_index[1].astype(jnp.int32)
  pad = EP - E
  srcp = jnp.concatenate([src, jnp.zeros((pad,), jnp.int32)])
  dstp = jnp.concatenate([dst, jnp.full((pad,), N, jnp.int32)])

  zeros_h = jnp.zeros((CH, H), jnp.float32)
  ones_h = jnp.ones((CH, H), jnp.float32)

  cntp = _make_sc_degree()(dstp, ones_h, zeros_h)

  h = x
  for wl, wr, b in ((Wl1, Wr1, b1), (Wl2, Wr2, b2), (Wl3, Wr3, b3),
                    (Wl4, Wr4, b4), (Wl5, Wr5, b5)):
    msg = _make_sc_segment_sum()(h, srcp, dstp, zeros_h)
    h = _tc_combine(msg, cntp, h, wl, wr, b)

  return _tc_pool(h, batch.astype(jnp.int32).reshape(N, 1), Wo, bo)


# speedup vs baseline: 1.1611x; 1.1611x over previous
import dataclasses
import functools

import jax
import jax.numpy as jnp
from jax import lax
from jax.experimental import pallas as pl
from jax.experimental.pallas import tpu as pltpu
from jax.experimental.pallas import tpu_sc as plsc

N = 10000
E = 320000
D = 128
H = 128
C = 64
G = 128

NC = 2
NS = 16
NW = NC * NS

CH = 128
WIN = 320
NPAD = NW * WIN
DUMMY = 328
ACC_ROWS = 336
EPS = 320256
KSHIFT = 14


@functools.cache
def _make_sc_segment_sum():
  mesh = plsc.VectorSubcoreMesh(core_axis_name="c", subcore_axis_name="s")
  cp = pltpu.CompilerParams()
  if "needs_layout_passes" in pltpu.CompilerParams.__dataclass_fields__:
    cp = dataclasses.replace(cp, needs_layout_passes=False)

  @functools.partial(
      pl.kernel,
      out_type=jax.ShapeDtypeStruct((NPAD, H), jnp.float32),
      mesh=mesh,
      compiler_params=cp,
      scratch_types=[
          pltpu.VMEM((16,), jnp.int32),
          pltpu.VMEM((16,), jnp.int32),
          pltpu.VMEM((CH,), jnp.int32),
          pltpu.VMEM((CH,), jnp.int32),
          pltpu.VMEM((CH, H), jnp.float32),
          pltpu.VMEM_SHARED((NS * ACC_ROWS, H), jnp.float32),
          pltpu.SemaphoreType.DMA,
      ],
  )
  def sc_segment_sum(h_hbm, src_hbm, dst_hbm, lo_hbm, nch_hbm, zeros_hbm,
                     out_hbm, lo_v, nch_v, src_v, dst_v, rows_v, acc, sem):
    cid = lax.axis_index("c")
    sid = lax.axis_index("s")
    wid = sid * NC + cid
    base = wid * WIN
    arow = sid * ACC_ROWS

    pltpu.sync_copy(zeros_hbm, acc.at[pl.ds(arow, CH)])
    pltpu.sync_copy(zeros_hbm, acc.at[pl.ds(arow + CH, CH)])
    pltpu.sync_copy(zeros_hbm.at[pl.ds(0, ACC_ROWS - 2 * CH)],
                    acc.at[pl.ds(arow + 2 * CH, ACC_ROWS - 2 * CH)])

    boff = pl.multiple_of(wid * 16, 16)
    pltpu.sync_copy(lo_hbm.at[pl.ds(boff, 16)], lo_v)
    pltpu.sync_copy(nch_hbm.at[pl.ds(boff, 16)], nch_v)
    lo = jnp.max(lo_v[...], axis=0)
    nch = jnp.max(nch_v[...], axis=0)

    def body(c, carry):
      off = pl.multiple_of(lo + c * CH, CH)
      pltpu.sync_copy(src_hbm.at[pl.ds(off, CH)], src_v)
      pltpu.sync_copy(dst_hbm.at[pl.ds(off, CH)], dst_v)
      pltpu.async_copy(h_hbm.at[src_v], rows_v, sem).wait()
      for j in range(CH // 16):
        d = dst_v[pl.ds(j * 16, 16)]
        t = d - base
        ok = (t >= 0) & (t < WIN)
        dst_v[pl.ds(j * 16, 16)] = jnp.where(ok, t + arow, DUMMY + arow)
      pltpu.sync_copy(rows_v, acc.at[dst_v], add=True)
      return carry

    lax.fori_loop(0, nch, body, 0)

    pltpu.sync_copy(acc.at[pl.ds(arow, WIN)], out_hbm.at[pl.ds(base, WIN)])

  return sc_segment_sum


_RB = 320
_NB = NPAD // _RB


def _combine_body(m_ref, rlo_ref, rhi_ref, h_ref, wl_ref, wr_ref, b_ref,
                  o_ref):
  cnt = (rhi_ref[...] - rlo_ref[...]).astype(jnp.float32)
  inv = 1.0 / jnp.maximum(cnt, 1.0)
  agg = m_ref[...] * inv
  z = (jnp.dot(agg, wl_ref[...], preferred_element_type=jnp.float32)
       + jnp.dot(h_ref[...], wr_ref[...], preferred_element_type=jnp.float32)
       + b_ref[...])
  o_ref[...] = jnp.maximum(z, 0.0)


def _tc_combine(msg, rlo, rhi, h, wl, wr, b):
  return pl.pallas_call(
      _combine_body,
      grid=(_NB,),
      in_specs=[
          pl.BlockSpec((_RB, H), lambda i: (i, 0)),
          pl.BlockSpec((_RB, 1), lambda i: (i, 0)),
          pl.BlockSpec((_RB, 1), lambda i: (i, 0)),
          pl.BlockSpec((_RB, H), lambda i: (i, 0)),
          pl.BlockSpec((H, H), lambda i: (0, 0)),
          pl.BlockSpec((H, H), lambda i: (0, 0)),
          pl.BlockSpec((1, H), lambda i: (0, 0)),
      ],
      out_specs=pl.BlockSpec((_RB, H), lambda i: (i, 0)),
      out_shape=jax.ShapeDtypeStruct((NPAD, H), jnp.float32),
  )(msg, rlo, rhi, h, wl, wr, b.reshape(1, H))


def _pool_body(h_ref, b_ref, wo_ref, bo_ref, o_ref, acc_ref, cacc_ref):
  i = pl.program_id(0)

  @pl.when(i == 0)
  def _():
    acc_ref[...] = jnp.zeros_like(acc_ref)
    cacc_ref[...] = jnp.zeros_like(cacc_ref)

  h = h_ref[...]
  bidx = b_ref[...]
  iota_g = lax.broadcasted_iota(jnp.int32, (_RB, G), 1)
  onehot = (bidx == iota_g).astype(jnp.float32)
  acc_ref[...] += lax.dot_general(
      onehot, h, (((0,), (0,)), ((), ())), preferred_element_type=jnp.float32)
  cacc_ref[...] += lax.dot_general(
      onehot, jnp.ones((_RB, G), jnp.float32), (((0,), (0,)), ((), ())),
      preferred_element_type=jnp.float32)

  @pl.when(i == _NB - 1)
  def _():
    cnt = jnp.maximum(cacc_ref[:, 0:1], 1.0)
    pooled = acc_ref[...] / cnt
    logits = (jnp.dot(pooled, wo_ref[...], preferred_element_type=jnp.float32)
              + bo_ref[...])
    m = jnp.max(logits, axis=1, keepdims=True)
    lse = jnp.log(jnp.sum(jnp.exp(logits - m), axis=1, keepdims=True)) + m
    o_ref[...] = logits - lse


def _tc_pool(h, batch2d, wo, bo):
  return pl.pallas_call(
      _pool_body,
      grid=(_NB,),
      in_specs=[
          pl.BlockSpec((_RB, H), lambda i: (i, 0)),
          pl.BlockSpec((_RB, 1), lambda i: (i, 0)),
          pl.BlockSpec((H, C), lambda i: (0, 0)),
          pl.BlockSpec((1, C), lambda i: (0, 0)),
      ],
      out_specs=pl.BlockSpec((G, C), lambda i: (0, 0)),
      out_shape=jax.ShapeDtypeStruct((G, C), jnp.float32),
      scratch_shapes=[
          pltpu.VMEM((G, H), jnp.float32),
          pltpu.VMEM((G, G), jnp.float32),
      ],
  )(h, batch2d, wo, bo.reshape(1, C))


def kernel(x, edge_index, batch, Wl1, Wr1, b1, Wl2, Wr2, b2, Wl3, Wr3, b3,
           Wl4, Wr4, b4, Wl5, Wr5, b5, Wo, bo):
  src = edge_index[0].astype(jnp.int32)
  dst = edge_index[1].astype(jnp.int32)

  keys = jnp.sort(dst * (1 << KSHIFT) + src)
  keys = jnp.concatenate(
      [keys, jnp.full((EPS - E,), NPAD << KSHIFT, jnp.int32)])
  srcs = keys & ((1 << KSHIFT) - 1)
  dsts = keys >> KSHIFT

  bounds = jnp.searchsorted(dsts, jnp.arange(0, NPAD + 1, WIN)).astype(
      jnp.int32)
  lo = bounds[:-1]
  hi = bounds[1:]
  lo128 = (lo // CH) * CH
  nch = (hi - lo128 + (CH - 1)) // CH
  lo_b = jnp.repeat(lo128, 16)
  nch_b = jnp.repeat(nch, 16)

  rp = jnp.searchsorted(dsts, jnp.arange(N + 1)).astype(jnp.int32)
  zpad = jnp.zeros((NPAD - N,), jnp.int32)
  rlo = jnp.concatenate([rp[:N], zpad]).reshape(NPAD, 1)
  rhi = jnp.concatenate([rp[1:], zpad]).reshape(NPAD, 1)

  zeros_h = jnp.zeros((CH, H), jnp.float32)

  seg = _make_sc_segment_sum()

  h = jnp.concatenate([x, jnp.zeros((NPAD - N, H), jnp.float32)])
  for wl, wr, b in ((Wl1, Wr1, b1), (Wl2, Wr2, b2), (Wl3, Wr3, b3),
                    (Wl4, Wr4, b4), (Wl5, Wr5, b5)):
    msg = seg(h, srcs, dsts, lo_b, nch_b, zeros_h)
    h = _tc_combine(msg, rlo, rhi, h, wl, wr, b)

  batchp = jnp.concatenate([batch.astype(jnp.int32),
                            jnp.full((NPAD - N,), G, jnp.int32)])
  return _tc_pool(h, batchp.reshape(NPAD, 1), Wo, bo)


# speedup vs baseline: 1.5511x; 1.3359x over previous
import dataclasses
import functools

import jax
import jax.numpy as jnp
from jax import lax
from jax.experimental import pallas as pl
from jax.experimental.pallas import tpu as pltpu
from jax.experimental.pallas import tpu_sc as plsc

N = 10000
E = 320000
D = 128
H = 128
C = 64
G = 128

NC = 2
NS = 16
NW = NC * NS

CH = 128
WIN = 320
NPAD = NW * WIN
DUMMY = 328
ACC_ROWS = 336
EPS = 320256
KSHIFT = 14


@functools.cache
def _make_sc_segment_sum():
  mesh = plsc.VectorSubcoreMesh(core_axis_name="c", subcore_axis_name="s")
  cp = pltpu.CompilerParams()
  if "needs_layout_passes" in pltpu.CompilerParams.__dataclass_fields__:
    cp = dataclasses.replace(cp, needs_layout_passes=False)

  @functools.partial(
      pl.kernel,
      out_type=jax.ShapeDtypeStruct((NPAD, H), jnp.float32),
      mesh=mesh,
      compiler_params=cp,
      scratch_types=[
          pltpu.VMEM((16,), jnp.int32),
          pltpu.VMEM((16,), jnp.int32),
          pltpu.VMEM((CH,), jnp.int32),
          pltpu.VMEM((CH,), jnp.int32),
          pltpu.VMEM((CH, H), jnp.float32),
          pltpu.VMEM_SHARED((NS * ACC_ROWS, H), jnp.float32),
          pltpu.SemaphoreType.DMA,
      ],
  )
  def sc_segment_sum(h_hbm, src_hbm, dst_hbm, lo_hbm, nch_hbm, zeros_hbm,
                     out_hbm, lo_v, nch_v, src_v, dst_v, rows_v, acc, sem):
    cid = lax.axis_index("c")
    sid = lax.axis_index("s")
    wid = sid * NC + cid
    base = wid * WIN
    arow = sid * ACC_ROWS

    pltpu.sync_copy(zeros_hbm, acc.at[pl.ds(arow, CH)])
    pltpu.sync_copy(zeros_hbm, acc.at[pl.ds(arow + CH, CH)])
    pltpu.sync_copy(zeros_hbm.at[pl.ds(0, ACC_ROWS - 2 * CH)],
                    acc.at[pl.ds(arow + 2 * CH, ACC_ROWS - 2 * CH)])

    boff = pl.multiple_of(wid * 16, 16)
    pltpu.sync_copy(lo_hbm.at[pl.ds(boff, 16)], lo_v)
    pltpu.sync_copy(nch_hbm.at[pl.ds(boff, 16)], nch_v)
    lo = jnp.max(lo_v[...], axis=0)
    nch = jnp.max(nch_v[...], axis=0)

    def body(c, carry):
      off = pl.multiple_of(lo + c * CH, CH)
      pltpu.sync_copy(src_hbm.at[pl.ds(off, CH)], src_v)
      pltpu.sync_copy(dst_hbm.at[pl.ds(off, CH)], dst_v)
      pltpu.async_copy(h_hbm.at[src_v], rows_v, sem).wait()
      for j in range(CH // 16):
        d = dst_v[pl.ds(j * 16, 16)]
        t = d - base
        ok = (t >= 0) & (t < WIN)
        dst_v[pl.ds(j * 16, 16)] = jnp.where(ok, t + arow, DUMMY + arow)
      pltpu.sync_copy(rows_v, acc.at[dst_v], add=True)
      return carry

    lax.fori_loop(0, nch, body, 0)

    pltpu.sync_copy(acc.at[pl.ds(arow, WIN)], out_hbm.at[pl.ds(base, WIN)])

  return sc_segment_sum


@functools.cache
def _make_sc_degree():
  mesh = plsc.VectorSubcoreMesh(core_axis_name="c", subcore_axis_name="s")
  cp = pltpu.CompilerParams()
  if "needs_layout_passes" in pltpu.CompilerParams.__dataclass_fields__:
    cp = dataclasses.replace(cp, needs_layout_passes=False)

  @functools.partial(
      pl.kernel,
      out_type=jax.ShapeDtypeStruct((NPAD, H), jnp.float32),
      mesh=mesh,
      compiler_params=cp,
      scratch_types=[
          pltpu.VMEM((16,), jnp.int32),
          pltpu.VMEM((16,), jnp.int32),
          pltpu.VMEM((CH,), jnp.int32),
          pltpu.VMEM((CH, H), jnp.float32),
          pltpu.VMEM_SHARED((NS * ACC_ROWS, H), jnp.float32),
      ],
  )
  def sc_degree(dst_hbm, lo_hbm, nch_hbm, ones_hbm, zeros_hbm,
                out_hbm, lo_v, nch_v, dst_v, ones_v, acc):
    cid = lax.axis_index("c")
    sid = lax.axis_index("s")
    wid = sid * NC + cid
    base = wid * WIN
    arow = sid * ACC_ROWS

    pltpu.sync_copy(ones_hbm, ones_v)
    pltpu.sync_copy(zeros_hbm, acc.at[pl.ds(arow, CH)])
    pltpu.sync_copy(zeros_hbm, acc.at[pl.ds(arow + CH, CH)])
    pltpu.sync_copy(zeros_hbm.at[pl.ds(0, ACC_ROWS - 2 * CH)],
                    acc.at[pl.ds(arow + 2 * CH, ACC_ROWS - 2 * CH)])

    boff = pl.multiple_of(wid * 16, 16)
    pltpu.sync_copy(lo_hbm.at[pl.ds(boff, 16)], lo_v)
    pltpu.sync_copy(nch_hbm.at[pl.ds(boff, 16)], nch_v)
    lo = jnp.max(lo_v[...], axis=0)
    nch = jnp.max(nch_v[...], axis=0)

    def body(c, carry):
      off = pl.multiple_of(lo + c * CH, CH)
      pltpu.sync_copy(dst_hbm.at[pl.ds(off, CH)], dst_v)
      for j in range(CH // 16):
        d = dst_v[pl.ds(j * 16, 16)]
        t = d - base
        ok = (t >= 0) & (t < WIN)
        dst_v[pl.ds(j * 16, 16)] = jnp.where(ok, t + arow, DUMMY + arow)
      pltpu.sync_copy(ones_v, acc.at[dst_v], add=True)
      return carry

    lax.fori_loop(0, nch, body, 0)

    pltpu.sync_copy(acc.at[pl.ds(arow, WIN)], out_hbm.at[pl.ds(base, WIN)])

  return sc_degree


_RB = 320
_NB = NPAD // _RB


def _combine_body(m_ref, c_ref, h_ref, wl_ref, wr_ref, b_ref, o_ref):
  cnt = c_ref[:, 0:1]
  inv = 1.0 / jnp.maximum(cnt, 1.0)
  agg = m_ref[...] * inv
  z = (jnp.dot(agg, wl_ref[...], preferred_element_type=jnp.float32)
       + jnp.dot(h_ref[...], wr_ref[...], preferred_element_type=jnp.float32)
       + b_ref[...])
  o_ref[...] = jnp.maximum(z, 0.0)


def _tc_combine(msg, cntp, h, wl, wr, b):
  return pl.pallas_call(
      _combine_body,
      grid=(_NB,),
      in_specs=[
          pl.BlockSpec((_RB, H), lambda i: (i, 0)),
          pl.BlockSpec((_RB, H), lambda i: (i, 0)),
          pl.BlockSpec((_RB, H), lambda i: (i, 0)),
          pl.BlockSpec((H, H), lambda i: (0, 0)),
          pl.BlockSpec((H, H), lambda i: (0, 0)),
          pl.BlockSpec((1, H), lambda i: (0, 0)),
      ],
      out_specs=pl.BlockSpec((_RB, H), lambda i: (i, 0)),
      out_shape=jax.ShapeDtypeStruct((NPAD, H), jnp.float32),
  )(msg, cntp, h, wl, wr, b.reshape(1, H))


def _pool_body(h_ref, b_ref, wo_ref, bo_ref, o_ref, acc_ref, cacc_ref):
  i = pl.program_id(0)

  @pl.when(i == 0)
  def _():
    acc_ref[...] = jnp.zeros_like(acc_ref)
    cacc_ref[...] = jnp.zeros_like(cacc_ref)

  h = h_ref[...]
  bidx = b_ref[...]
  iota_g = lax.broadcasted_iota(jnp.int32, (_RB, G), 1)
  onehot = (bidx == iota_g).astype(jnp.float32)
  acc_ref[...] += lax.dot_general(
      onehot, h, (((0,), (0,)), ((), ())), preferred_element_type=jnp.float32)
  cacc_ref[...] += lax.dot_general(
      onehot, jnp.ones((_RB, G), jnp.float32), (((0,), (0,)), ((), ())),
      preferred_element_type=jnp.float32)

  @pl.when(i == _NB - 1)
  def _():
    cnt = jnp.maximum(cacc_ref[:, 0:1], 1.0)
    pooled = acc_ref[...] / cnt
    logits = (jnp.dot(pooled, wo_ref[...], preferred_element_type=jnp.float32)
              + bo_ref[...])
    m = jnp.max(logits, axis=1, keepdims=True)
    lse = jnp.log(jnp.sum(jnp.exp(logits - m), axis=1, keepdims=True)) + m
    o_ref[...] = logits - lse


def _tc_pool(h, batch2d, wo, bo):
  return pl.pallas_call(
      _pool_body,
      grid=(_NB,),
      in_specs=[
          pl.BlockSpec((_RB, H), lambda i: (i, 0)),
          pl.BlockSpec((_RB, 1), lambda i: (i, 0)),
          pl.BlockSpec((H, C), lambda i: (0, 0)),
          pl.BlockSpec((1, C), lambda i: (0, 0)),
      ],
      out_specs=pl.BlockSpec((G, C), lambda i: (0, 0)),
      out_shape=jax.ShapeDtypeStruct((G, C), jnp.float32),
      scratch_shapes=[
          pltpu.VMEM((G, H), jnp.float32),
          pltpu.VMEM((G, G), jnp.float32),
      ],
  )(h, batch2d, wo, bo.reshape(1, C))


def kernel(x, edge_index, batch, Wl1, Wr1, b1, Wl2, Wr2, b2, Wl3, Wr3, b3,
           Wl4, Wr4, b4, Wl5, Wr5, b5, Wo, bo):
  src = edge_index[0].astype(jnp.int32)
  dst = edge_index[1].astype(jnp.int32)

  keys = jnp.sort(dst * (1 << KSHIFT) + src)
  keys = jnp.concatenate(
      [keys, jnp.full((EPS - E,), NPAD << KSHIFT, jnp.int32)])
  srcs = keys & ((1 << KSHIFT) - 1)
  dsts = keys >> KSHIFT

  bounds = jnp.searchsorted(dsts, jnp.arange(0, NPAD + 1, WIN)).astype(
      jnp.int32)
  lo = bounds[:-1]
  hi = bounds[1:]
  lo128 = (lo // CH) * CH
  nch = (hi - lo128 + (CH - 1)) // CH
  lo_b = jnp.repeat(lo128, 16)
  nch_b = jnp.repeat(nch, 16)

  zeros_h = jnp.zeros((CH, H), jnp.float32)
  ones_h = jnp.ones((CH, H), jnp.float32)

  cntp = _make_sc_degree()(dsts, lo_b, nch_b, ones_h, zeros_h)

  seg = _make_sc_segment_sum()

  h = jnp.concatenate([x, jnp.zeros((NPAD - N, H), jnp.float32)])
  for wl, wr, b in ((Wl1, Wr1, b1), (Wl2, Wr2, b2), (Wl3, Wr3, b3),
                    (Wl4, Wr4, b4), (Wl5, Wr5, b5)):
    msg = seg(h, srcs, dsts, lo_b, nch_b, zeros_h)
    h = _tc_combine(msg, cntp, h, wl, wr, b)

  batchp = jnp.concatenate([batch.astype(jnp.int32),
                            jnp.full((NPAD - N,), G, jnp.int32)])
  return _tc_pool(h, batchp.reshape(NPAD, 1), Wo, bo)


# speedup vs baseline: 2.2323x; 1.4392x over previous
import dataclasses
import functools

import jax
import jax.numpy as jnp
from jax import lax
from jax.experimental import pallas as pl
from jax.experimental.pallas import tpu as pltpu
from jax.experimental.pallas import tpu_sc as plsc

N = 10000
E = 320000
D = 128
H = 128
C = 64
G = 128

NC = 2
NS = 16
NW = NC * NS

CH = 128
WIN = 320
NPAD = NW * WIN
DUMMY = 328
ACC_ROWS = 336
EPS = 320512
KSHIFT = 14


@functools.cache
def _make_sc_segment_sum():
  mesh = plsc.VectorSubcoreMesh(core_axis_name="c", subcore_axis_name="s")
  cp = pltpu.CompilerParams()
  if "needs_layout_passes" in pltpu.CompilerParams.__dataclass_fields__:
    cp = dataclasses.replace(cp, needs_layout_passes=False)

  @functools.partial(
      pl.kernel,
      out_type=jax.ShapeDtypeStruct((NPAD, H), jnp.float32),
      mesh=mesh,
      compiler_params=cp,
      scratch_types=[
          pltpu.VMEM((16,), jnp.int32),
          pltpu.VMEM((16,), jnp.int32),
          pltpu.VMEM((CH,), jnp.int32),
          pltpu.VMEM((CH,), jnp.int32),
          pltpu.VMEM((CH,), jnp.int32),
          pltpu.VMEM((CH,), jnp.int32),
          pltpu.VMEM((CH, H), jnp.float32),
          pltpu.VMEM((CH, H), jnp.float32),
          pltpu.VMEM_SHARED((NS * ACC_ROWS, H), jnp.float32),
          pltpu.SemaphoreType.DMA,
          pltpu.SemaphoreType.DMA,
      ],
  )
  def sc_segment_sum(h_hbm, src_hbm, dst_hbm, lo_hbm, nch_hbm, zeros_hbm,
                     out_hbm, lo_v, nch_v, src_a, dst_a, src_b, dst_b,
                     rows_a, rows_b, acc, sem_g, sem_i):
    cid = lax.axis_index("c")
    sid = lax.axis_index("s")
    wid = sid * NC + cid
    base = wid * WIN
    arow = sid * ACC_ROWS

    pltpu.sync_copy(zeros_hbm, acc.at[pl.ds(arow, CH)])
    pltpu.sync_copy(zeros_hbm, acc.at[pl.ds(arow + CH, CH)])
    pltpu.sync_copy(zeros_hbm.at[pl.ds(0, ACC_ROWS - 2 * CH)],
                    acc.at[pl.ds(arow + 2 * CH, ACC_ROWS - 2 * CH)])

    boff = pl.multiple_of(wid * 16, 16)
    pltpu.sync_copy(lo_hbm.at[pl.ds(boff, 16)], lo_v)
    pltpu.sync_copy(nch_hbm.at[pl.ds(boff, 16)], nch_v)
    lo = jnp.max(lo_v[...], axis=0)
    npairs = jnp.max(nch_v[...], axis=0)

    def issue_idx(c, src_v, dst_v):
      off = pl.multiple_of(lo + c * CH, CH)
      pltpu.async_copy(src_hbm.at[pl.ds(off, CH)], src_v, sem_i)
      pltpu.async_copy(dst_hbm.at[pl.ds(off, CH)], dst_v, sem_i)

    def wait_idx(src_v, dst_v):
      pltpu.make_async_copy(src_hbm.at[pl.ds(0, CH)], src_v, sem_i).wait()
      pltpu.make_async_copy(dst_hbm.at[pl.ds(0, CH)], dst_v, sem_i).wait()

    def issue_gather(src_v, rows_v):
      pltpu.async_copy(h_hbm.at[src_v], rows_v, sem_g)

    def wait_gather(src_v, rows_v):
      pltpu.make_async_copy(h_hbm.at[src_v], rows_v, sem_g).wait()

    def remap_scatter(dst_v, rows_v):
      for j in range(CH // 16):
        d = dst_v[pl.ds(j * 16, 16)]
        t = d - base
        ok = (t >= 0) & (t < WIN)
        dst_v[pl.ds(j * 16, 16)] = jnp.where(ok, t + arow, DUMMY + arow)
      pltpu.sync_copy(rows_v, acc.at[dst_v], add=True)

    issue_idx(0, src_a, dst_a)
    wait_idx(src_a, dst_a)
    issue_gather(src_a, rows_a)
    issue_idx(1, src_b, dst_b)

    def body(p, carry):
      a = 2 * p
      wait_gather(src_a, rows_a)
      wait_idx(src_b, dst_b)
      issue_gather(src_b, rows_b)
      remap_scatter(dst_a, rows_a)
      issue_idx(a + 2, src_a, dst_a)
      wait_gather(src_b, rows_b)
      wait_idx(src_a, dst_a)
      issue_gather(src_a, rows_a)
      remap_scatter(dst_b, rows_b)
      issue_idx(a + 3, src_b, dst_b)
      return carry

    lax.fori_loop(0, npairs, body, 0)

    wait_gather(src_a, rows_a)
    wait_idx(src_b, dst_b)

    pltpu.sync_copy(acc.at[pl.ds(arow, WIN)], out_hbm.at[pl.ds(base, WIN)])

  return sc_segment_sum


@functools.cache
def _make_sc_degree():
  mesh = plsc.VectorSubcoreMesh(core_axis_name="c", subcore_axis_name="s")
  cp = pltpu.CompilerParams()
  if "needs_layout_passes" in pltpu.CompilerParams.__dataclass_fields__:
    cp = dataclasses.replace(cp, needs_layout_passes=False)

  @functools.partial(
      pl.kernel,
      out_type=jax.ShapeDtypeStruct((NPAD, H), jnp.float32),
      mesh=mesh,
      compiler_params=cp,
      scratch_types=[
          pltpu.VMEM((16,), jnp.int32),
          pltpu.VMEM((16,), jnp.int32),
          pltpu.VMEM((CH,), jnp.int32),
          pltpu.VMEM((CH, H), jnp.float32),
          pltpu.VMEM_SHARED((NS * ACC_ROWS, H), jnp.float32),
      ],
  )
  def sc_degree(dst_hbm, lo_hbm, nch_hbm, ones_hbm, zeros_hbm,
                out_hbm, lo_v, nch_v, dst_v, ones_v, acc):
    cid = lax.axis_index("c")
    sid = lax.axis_index("s")
    wid = sid * NC + cid
    base = wid * WIN
    arow = sid * ACC_ROWS

    pltpu.sync_copy(ones_hbm, ones_v)
    pltpu.sync_copy(zeros_hbm, acc.at[pl.ds(arow, CH)])
    pltpu.sync_copy(zeros_hbm, acc.at[pl.ds(arow + CH, CH)])
    pltpu.sync_copy(zeros_hbm.at[pl.ds(0, ACC_ROWS - 2 * CH)],
                    acc.at[pl.ds(arow + 2 * CH, ACC_ROWS - 2 * CH)])

    boff = pl.multiple_of(wid * 16, 16)
    pltpu.sync_copy(lo_hbm.at[pl.ds(boff, 16)], lo_v)
    pltpu.sync_copy(nch_hbm.at[pl.ds(boff, 16)], nch_v)
    lo = jnp.max(lo_v[...], axis=0)
    nch = jnp.max(nch_v[...], axis=0)

    def body(c, carry):
      off = pl.multiple_of(lo + c * CH, CH)
      pltpu.sync_copy(dst_hbm.at[pl.ds(off, CH)], dst_v)
      for j in range(CH // 16):
        d = dst_v[pl.ds(j * 16, 16)]
        t = d - base
        ok = (t >= 0) & (t < WIN)
        dst_v[pl.ds(j * 16, 16)] = jnp.where(ok, t + arow, DUMMY + arow)
      pltpu.sync_copy(ones_v, acc.at[dst_v], add=True)
      return carry

    lax.fori_loop(0, nch, body, 0)

    pltpu.sync_copy(acc.at[pl.ds(arow, WIN)], out_hbm.at[pl.ds(base, WIN)])

  return sc_degree


_RB = 320
_NB = NPAD // _RB


def _combine_body(m_ref, c_ref, h_ref, wl_ref, wr_ref, b_ref, o_ref):
  cnt = c_ref[:, 0:1]
  inv = 1.0 / jnp.maximum(cnt, 1.0)
  agg = m_ref[...] * inv
  z = (jnp.dot(agg, wl_ref[...], preferred_element_type=jnp.float32)
       + jnp.dot(h_ref[...], wr_ref[...], preferred_element_type=jnp.float32)
       + b_ref[...])
  o_ref[...] = jnp.maximum(z, 0.0)


def _tc_combine(msg, cntp, h, wl, wr, b):
  return pl.pallas_call(
      _combine_body,
      grid=(_NB,),
      in_specs=[
          pl.BlockSpec((_RB, H), lambda i: (i, 0)),
          pl.BlockSpec((_RB, H), lambda i: (i, 0)),
          pl.BlockSpec((_RB, H), lambda i: (i, 0)),
          pl.BlockSpec((H, H), lambda i: (0, 0)),
          pl.BlockSpec((H, H), lambda i: (0, 0)),
          pl.BlockSpec((1, H), lambda i: (0, 0)),
      ],
      out_specs=pl.BlockSpec((_RB, H), lambda i: (i, 0)),
      out_shape=jax.ShapeDtypeStruct((NPAD, H), jnp.float32),
  )(msg, cntp, h, wl, wr, b.reshape(1, H))


def _pool_body(h_ref, b_ref, wo_ref, bo_ref, o_ref, acc_ref, cacc_ref):
  i = pl.program_id(0)

  @pl.when(i == 0)
  def _():
    acc_ref[...] = jnp.zeros_like(acc_ref)
    cacc_ref[...] = jnp.zeros_like(cacc_ref)

  h = h_ref[...]
  bidx = b_ref[...]
  iota_g = lax.broadcasted_iota(jnp.int32, (_RB, G), 1)
  onehot = (bidx == iota_g).astype(jnp.float32)
  acc_ref[...] += lax.dot_general(
      onehot, h, (((0,), (0,)), ((), ())), preferred_element_type=jnp.float32)
  cacc_ref[...] += lax.dot_general(
      onehot, jnp.ones((_RB, G), jnp.float32), (((0,), (0,)), ((), ())),
      preferred_element_type=jnp.float32)

  @pl.when(i == _NB - 1)
  def _():
    cnt = jnp.maximum(cacc_ref[:, 0:1], 1.0)
    pooled = acc_ref[...] / cnt
    logits = (jnp.dot(pooled, wo_ref[...], preferred_element_type=jnp.float32)
              + bo_ref[...])
    m = jnp.max(logits, axis=1, keepdims=True)
    lse = jnp.log(jnp.sum(jnp.exp(logits - m), axis=1, keepdims=True)) + m
    o_ref[...] = logits - lse


def _tc_pool(h, batch2d, wo, bo):
  return pl.pallas_call(
      _pool_body,
      grid=(_NB,),
      in_specs=[
          pl.BlockSpec((_RB, H), lambda i: (i, 0)),
          pl.BlockSpec((_RB, 1), lambda i: (i, 0)),
          pl.BlockSpec((H, C), lambda i: (0, 0)),
          pl.BlockSpec((1, C), lambda i: (0, 0)),
      ],
      out_specs=pl.BlockSpec((G, C), lambda i: (0, 0)),
      out_shape=jax.ShapeDtypeStruct((G, C), jnp.float32),
      scratch_shapes=[
          pltpu.VMEM((G, H), jnp.float32),
          pltpu.VMEM((G, G), jnp.float32),
      ],
  )(h, batch2d, wo, bo.reshape(1, C))


def kernel(x, edge_index, batch, Wl1, Wr1, b1, Wl2, Wr2, b2, Wl3, Wr3, b3,
           Wl4, Wr4, b4, Wl5, Wr5, b5, Wo, bo):
  src = edge_index[0].astype(jnp.int32)
  dst = edge_index[1].astype(jnp.int32)

  keys = jnp.sort(dst * (1 << KSHIFT) + src)
  keys = jnp.concatenate(
      [keys, jnp.full((EPS - E,), NPAD << KSHIFT, jnp.int32)])
  srcs = keys & ((1 << KSHIFT) - 1)
  dsts = keys >> KSHIFT

  bounds = jnp.searchsorted(dsts, jnp.arange(0, NPAD + 1, WIN)).astype(
      jnp.int32)
  lo = bounds[:-1]
  hi = bounds[1:]
  lo128 = (lo // CH) * CH
  nch = (hi - lo128 + (CH - 1)) // CH
  npairs = (nch + 1) // 2
  lo_b = jnp.repeat(lo128, 16)
  nch_b = jnp.repeat(nch, 16)
  npairs_b = jnp.repeat(npairs, 16)

  zeros_h = jnp.zeros((CH, H), jnp.float32)
  ones_h = jnp.ones((CH, H), jnp.float32)

  cntp = _make_sc_degree()(dsts, lo_b, nch_b, ones_h, zeros_h)

  seg = _make_sc_segment_sum()

  h = jnp.concatenate([x, jnp.zeros((NPAD - N, H), jnp.float32)])
  for wl, wr, b in ((Wl1, Wr1, b1), (Wl2, Wr2, b2), (Wl3, Wr3, b3),
                    (Wl4, Wr4, b4), (Wl5, Wr5, b5)):
    msg = seg(h, srcs, dsts, lo_b, npairs_b, zeros_h)
    h = _tc_combine(msg, cntp, h, wl, wr, b)

  batchp = jnp.concatenate([batch.astype(jnp.int32),
                            jnp.full((NPAD - N,), G, jnp.int32)])
  return _tc_pool(h, batchp.reshape(NPAD, 1), Wo, bo)


# speedup vs baseline: 2.2338x; 1.0007x over previous
import dataclasses
import functools

import jax
import jax.numpy as jnp
from jax import lax
from jax.experimental import pallas as pl
from jax.experimental.pallas import tpu as pltpu
from jax.experimental.pallas import tpu_sc as plsc

N = 10000
E = 320000
D = 128
H = 128
C = 64
G = 128

NC = 2
NS = 16
NW = NC * NS

CH = 128
WIN = 320
NPAD = NW * WIN
DUMMY = 328
ACC_ROWS = 336
EPS = 320512
KSHIFT = 14


@functools.cache
def _make_sc_segment_sum():
  mesh = plsc.VectorSubcoreMesh(core_axis_name="c", subcore_axis_name="s")
  cp = pltpu.CompilerParams()
  if "needs_layout_passes" in pltpu.CompilerParams.__dataclass_fields__:
    cp = dataclasses.replace(cp, needs_layout_passes=False)

  @functools.partial(
      pl.kernel,
      out_type=jax.ShapeDtypeStruct((NC, NS * WIN, H), jnp.float32),
      mesh=mesh,
      compiler_params=cp,
      scratch_types=[
          pltpu.VMEM((16,), jnp.int32),
          pltpu.VMEM((16,), jnp.int32),
          pltpu.VMEM((CH,), jnp.int32),
          pltpu.VMEM((CH,), jnp.int32),
          pltpu.VMEM((CH,), jnp.int32),
          pltpu.VMEM((CH,), jnp.int32),
          pltpu.VMEM((CH, H), jnp.float32),
          pltpu.VMEM((CH, H), jnp.float32),
          pltpu.VMEM_SHARED((NS * ACC_ROWS, H), jnp.float32),
          pltpu.SemaphoreType.DMA,
          pltpu.SemaphoreType.DMA,
      ],
  )
  def sc_segment_sum(h_hbm, src_hbm, dst_hbm, lo_hbm, nch_hbm, zeros_hbm,
                     out_hbm, lo_v, nch_v, src_a, dst_a, src_b, dst_b,
                     rows_a, rows_b, acc, sem_g, sem_i):
    cid = lax.axis_index("c")
    sid = lax.axis_index("s")
    wid = cid * NS + sid
    base = wid * WIN
    arow = sid * ACC_ROWS

    pltpu.sync_copy(zeros_hbm, acc.at[pl.ds(arow, CH)])
    pltpu.sync_copy(zeros_hbm, acc.at[pl.ds(arow + CH, CH)])
    pltpu.sync_copy(zeros_hbm.at[pl.ds(0, ACC_ROWS - 2 * CH)],
                    acc.at[pl.ds(arow + 2 * CH, ACC_ROWS - 2 * CH)])

    boff = pl.multiple_of(wid * 16, 16)
    pltpu.sync_copy(lo_hbm.at[pl.ds(boff, 16)], lo_v)
    pltpu.sync_copy(nch_hbm.at[pl.ds(boff, 16)], nch_v)
    lo = jnp.max(lo_v[...], axis=0)
    npairs = jnp.max(nch_v[...], axis=0)

    def issue_idx(c, src_v, dst_v):
      off = pl.multiple_of(lo + c * CH, CH)
      pltpu.async_copy(src_hbm.at[pl.ds(off, CH)], src_v, sem_i)
      pltpu.async_copy(dst_hbm.at[pl.ds(off, CH)], dst_v, sem_i)

    def wait_idx(src_v, dst_v):
      pltpu.make_async_copy(src_hbm.at[pl.ds(0, CH)], src_v, sem_i).wait()
      pltpu.make_async_copy(dst_hbm.at[pl.ds(0, CH)], dst_v, sem_i).wait()

    def issue_gather(src_v, rows_v):
      pltpu.async_copy(h_hbm.at[src_v], rows_v, sem_g)

    def wait_gather(src_v, rows_v):
      pltpu.make_async_copy(h_hbm.at[src_v], rows_v, sem_g).wait()

    def remap_scatter(dst_v, rows_v):
      for j in range(CH // 16):
        d = dst_v[pl.ds(j * 16, 16)]
        t = d - base
        ok = (t >= 0) & (t < WIN)
        dst_v[pl.ds(j * 16, 16)] = jnp.where(ok, t + arow, DUMMY + arow)
      pltpu.sync_copy(rows_v, acc.at[dst_v], add=True)

    issue_idx(0, src_a, dst_a)
    wait_idx(src_a, dst_a)
    issue_gather(src_a, rows_a)
    issue_idx(1, src_b, dst_b)

    def body(p, carry):
      a = 2 * p
      wait_gather(src_a, rows_a)
      wait_idx(src_b, dst_b)
      issue_gather(src_b, rows_b)
      remap_scatter(dst_a, rows_a)
      issue_idx(a + 2, src_a, dst_a)
      wait_gather(src_b, rows_b)
      wait_idx(src_a, dst_a)
      issue_gather(src_a, rows_a)
      remap_scatter(dst_b, rows_b)
      issue_idx(a + 3, src_b, dst_b)
      return carry

    lax.fori_loop(0, npairs, body, 0)

    wait_gather(src_a, rows_a)
    wait_idx(src_b, dst_b)

    pltpu.sync_copy(acc.at[pl.ds(arow, WIN)],
                    out_hbm.at[cid].at[pl.ds(sid * WIN, WIN)])

  return sc_segment_sum


@functools.cache
def _make_sc_degree():
  mesh = plsc.VectorSubcoreMesh(core_axis_name="c", subcore_axis_name="s")
  cp = pltpu.CompilerParams()
  if "needs_layout_passes" in pltpu.CompilerParams.__dataclass_fields__:
    cp = dataclasses.replace(cp, needs_layout_passes=False)

  @functools.partial(
      pl.kernel,
      out_type=jax.ShapeDtypeStruct((NC, NS * WIN, H), jnp.float32),
      mesh=mesh,
      compiler_params=cp,
      scratch_types=[
          pltpu.VMEM((16,), jnp.int32),
          pltpu.VMEM((16,), jnp.int32),
          pltpu.VMEM((CH,), jnp.int32),
          pltpu.VMEM((CH, H), jnp.float32),
          pltpu.VMEM_SHARED((NS * ACC_ROWS, H), jnp.float32),
      ],
  )
  def sc_degree(dst_hbm, lo_hbm, nch_hbm, ones_hbm, zeros_hbm,
                out_hbm, lo_v, nch_v, dst_v, ones_v, acc):
    cid = lax.axis_index("c")
    sid = lax.axis_index("s")
    wid = cid * NS + sid
    base = wid * WIN
    arow = sid * ACC_ROWS

    pltpu.sync_copy(ones_hbm, ones_v)
    pltpu.sync_copy(zeros_hbm, acc.at[pl.ds(arow, CH)])
    pltpu.sync_copy(zeros_hbm, acc.at[pl.ds(arow + CH, CH)])
    pltpu.sync_copy(zeros_hbm.at[pl.ds(0, ACC_ROWS - 2 * CH)],
                    acc.at[pl.ds(arow + 2 * CH, ACC_ROWS - 2 * CH)])

    boff = pl.multiple_of(wid * 16, 16)
    pltpu.sync_copy(lo_hbm.at[pl.ds(boff, 16)], lo_v)
    pltpu.sync_copy(nch_hbm.at[pl.ds(boff, 16)], nch_v)
    lo = jnp.max(lo_v[...], axis=0)
    nch = jnp.max(nch_v[...], axis=0)

    def body(c, carry):
      off = pl.multiple_of(lo + c * CH, CH)
      pltpu.sync_copy(dst_hbm.at[pl.ds(off, CH)], dst_v)
      for j in range(CH // 16):
        d = dst_v[pl.ds(j * 16, 16)]
        t = d - base
        ok = (t >= 0) & (t < WIN)
        dst_v[pl.ds(j * 16, 16)] = jnp.where(ok, t + arow, DUMMY + arow)
      pltpu.sync_copy(ones_v, acc.at[dst_v], add=True)
      return carry

    lax.fori_loop(0, nch, body, 0)

    pltpu.sync_copy(acc.at[pl.ds(arow, WIN)],
                    out_hbm.at[cid].at[pl.ds(sid * WIN, WIN)])

  return sc_degree


_RB = 320
_NB = NPAD // _RB


def _combine_body(m_ref, c_ref, h_ref, wl_ref, wr_ref, b_ref, o_ref):
  cnt = c_ref[:, 0:1]
  inv = 1.0 / jnp.maximum(cnt, 1.0)
  agg = m_ref[...] * inv
  z = (jnp.dot(agg, wl_ref[...], preferred_element_type=jnp.float32)
       + jnp.dot(h_ref[...], wr_ref[...], preferred_element_type=jnp.float32)
       + b_ref[...])
  o_ref[...] = jnp.maximum(z, 0.0)


def _tc_combine(msg, cntp, h, wl, wr, b):
  return pl.pallas_call(
      _combine_body,
      grid=(_NB,),
      in_specs=[
          pl.BlockSpec((_RB, H), lambda i: (i, 0)),
          pl.BlockSpec((_RB, H), lambda i: (i, 0)),
          pl.BlockSpec((_RB, H), lambda i: (i, 0)),
          pl.BlockSpec((H, H), lambda i: (0, 0)),
          pl.BlockSpec((H, H), lambda i: (0, 0)),
          pl.BlockSpec((1, H), lambda i: (0, 0)),
      ],
      out_specs=pl.BlockSpec((_RB, H), lambda i: (i, 0)),
      out_shape=jax.ShapeDtypeStruct((NPAD, H), jnp.float32),
  )(msg, cntp, h, wl, wr, b.reshape(1, H))


def _pool_body(h_ref, b_ref, wo_ref, bo_ref, o_ref, acc_ref, cacc_ref):
  i = pl.program_id(0)

  @pl.when(i == 0)
  def _():
    acc_ref[...] = jnp.zeros_like(acc_ref)
    cacc_ref[...] = jnp.zeros_like(cacc_ref)

  h = h_ref[...]
  bidx = b_ref[...]
  iota_g = lax.broadcasted_iota(jnp.int32, (_RB, G), 1)
  onehot = (bidx == iota_g).astype(jnp.float32)
  acc_ref[...] += lax.dot_general(
      onehot, h, (((0,), (0,)), ((), ())), preferred_element_type=jnp.float32)
  cacc_ref[...] += lax.dot_general(
      onehot, jnp.ones((_RB, G), jnp.float32), (((0,), (0,)), ((), ())),
      preferred_element_type=jnp.float32)

  @pl.when(i == _NB - 1)
  def _():
    cnt = jnp.maximum(cacc_ref[:, 0:1], 1.0)
    pooled = acc_ref[...] / cnt
    logits = (jnp.dot(pooled, wo_ref[...], preferred_element_type=jnp.float32)
              + bo_ref[...])
    m = jnp.max(logits, axis=1, keepdims=True)
    lse = jnp.log(jnp.sum(jnp.exp(logits - m), axis=1, keepdims=True)) + m
    o_ref[...] = logits - lse


def _tc_pool(h, batch2d, wo, bo):
  return pl.pallas_call(
      _pool_body,
      grid=(_NB,),
      in_specs=[
          pl.BlockSpec((_RB, H), lambda i: (i, 0)),
          pl.BlockSpec((_RB, 1), lambda i: (i, 0)),
          pl.BlockSpec((H, C), lambda i: (0, 0)),
          pl.BlockSpec((1, C), lambda i: (0, 0)),
      ],
      out_specs=pl.BlockSpec((G, C), lambda i: (0, 0)),
      out_shape=jax.ShapeDtypeStruct((G, C), jnp.float32),
      scratch_shapes=[
          pltpu.VMEM((G, H), jnp.float32),
          pltpu.VMEM((G, G), jnp.float32),
      ],
  )(h, batch2d, wo, bo.reshape(1, C))


def kernel(x, edge_index, batch, Wl1, Wr1, b1, Wl2, Wr2, b2, Wl3, Wr3, b3,
           Wl4, Wr4, b4, Wl5, Wr5, b5, Wo, bo):
  src = edge_index[0].astype(jnp.int32)
  dst = edge_index[1].astype(jnp.int32)

  keys = jnp.sort(dst * (1 << KSHIFT) + src)
  keys = jnp.concatenate(
      [keys, jnp.full((EPS - E,), NPAD << KSHIFT, jnp.int32)])
  srcs = keys & ((1 << KSHIFT) - 1)
  dsts = keys >> KSHIFT

  bounds = jnp.searchsorted(dsts, jnp.arange(0, NPAD + 1, WIN)).astype(
      jnp.int32)
  lo = bounds[:-1]
  hi = bounds[1:]
  lo128 = (lo // CH) * CH
  nch = (hi - lo128 + (CH - 1)) // CH
  npairs = (nch + 1) // 2
  lo_b = jnp.repeat(lo128, 16)
  nch_b = jnp.repeat(nch, 16)
  npairs_b = jnp.repeat(npairs, 16)

  zeros_h = jnp.zeros((CH, H), jnp.float32)
  ones_h = jnp.ones((CH, H), jnp.float32)

  cntp = _make_sc_degree()(dsts, lo_b, nch_b, ones_h, zeros_h).reshape(NPAD, H)

  seg = _make_sc_segment_sum()

  h = jnp.concatenate([x, jnp.zeros((NPAD - N, H), jnp.float32)])
  for wl, wr, b in ((Wl1, Wr1, b1), (Wl2, Wr2, b2), (Wl3, Wr3, b3),
                    (Wl4, Wr4, b4), (Wl5, Wr5, b5)):
    msg = seg(h, srcs, dsts, lo_b, npairs_b, zeros_h).reshape(NPAD, H)
    h = _tc_combine(msg, cntp, h, wl, wr, b)

  batchp = jnp.concatenate([batch.astype(jnp.int32),
                            jnp.full((NPAD - N,), G, jnp.int32)])
  return _tc_pool(h, batchp.reshape(NPAD, 1), Wo, bo)


# speedup vs baseline: 2.2783x; 1.0199x over previous
import dataclasses
import functools

import jax
import jax.numpy as jnp
from jax import lax
from jax.experimental import pallas as pl
from jax.experimental.pallas import tpu as pltpu
from jax.experimental.pallas import tpu_sc as plsc

N = 10000
E = 320000
D = 128
H = 128
C = 64
G = 128

NC = 2
NS = 16
NW = NC * NS

CH = 128
WIN = 320
NPAD = NW * WIN
DUMMY = 328
ACC_ROWS = 336
EPS = 320512
KSHIFT = 14


@functools.cache
def _make_sc_segment_sum():
  mesh = plsc.VectorSubcoreMesh(core_axis_name="c", subcore_axis_name="s")
  cp = pltpu.CompilerParams()
  if "needs_layout_passes" in pltpu.CompilerParams.__dataclass_fields__:
    cp = dataclasses.replace(cp, needs_layout_passes=False)

  @functools.partial(
      pl.kernel,
      out_type=jax.ShapeDtypeStruct((NC, NS * WIN, H), jnp.float32),
      mesh=mesh,
      compiler_params=cp,
      scratch_types=[
          pltpu.VMEM((16,), jnp.int32),
          pltpu.VMEM((16,), jnp.int32),
          pltpu.VMEM((CH,), jnp.int32),
          pltpu.VMEM((CH,), jnp.int32),
          pltpu.VMEM((CH,), jnp.int32),
          pltpu.VMEM((CH,), jnp.int32),
          pltpu.VMEM((CH, H), jnp.float32),
          pltpu.VMEM((CH, H), jnp.float32),
          pltpu.VMEM_SHARED((NS * ACC_ROWS, H), jnp.float32),
          pltpu.SemaphoreType.DMA,
          pltpu.SemaphoreType.DMA,
      ],
  )
  def sc_segment_sum(h_hbm, src_hbm, dst_hbm, lo_hbm, nch_hbm, zeros_hbm,
                     out_hbm, lo_v, nch_v, src_a, dst_a, src_b, dst_b,
                     rows_a, rows_b, acc, sem_g, sem_i):
    cid = lax.axis_index("c")
    sid = lax.axis_index("s")
    wid = cid * NS + sid
    base = wid * WIN
    arow = sid * ACC_ROWS

    pltpu.sync_copy(zeros_hbm, acc.at[pl.ds(arow, CH)])
    pltpu.sync_copy(zeros_hbm, acc.at[pl.ds(arow + CH, CH)])
    pltpu.sync_copy(zeros_hbm.at[pl.ds(0, ACC_ROWS - 2 * CH)],
                    acc.at[pl.ds(arow + 2 * CH, ACC_ROWS - 2 * CH)])

    boff = pl.multiple_of(wid * 16, 16)
    pltpu.sync_copy(lo_hbm.at[pl.ds(boff, 16)], lo_v)
    pltpu.sync_copy(nch_hbm.at[pl.ds(boff, 16)], nch_v)
    lo = jnp.max(lo_v[...], axis=0)
    npairs = jnp.max(nch_v[...], axis=0)

    def issue_idx(c, src_v, dst_v):
      off = pl.multiple_of(lo + c * CH, CH)
      pltpu.async_copy(src_hbm.at[pl.ds(off, CH)], src_v, sem_i)
      pltpu.async_copy(dst_hbm.at[pl.ds(off, CH)], dst_v, sem_i)

    def wait_idx(src_v, dst_v):
      pltpu.make_async_copy(src_hbm.at[pl.ds(0, CH)], src_v, sem_i).wait()
      pltpu.make_async_copy(dst_hbm.at[pl.ds(0, CH)], dst_v, sem_i).wait()

    def issue_gather(src_v, rows_v):
      pltpu.async_copy(h_hbm.at[src_v], rows_v, sem_g)

    def wait_gather(src_v, rows_v):
      pltpu.make_async_copy(h_hbm.at[src_v], rows_v, sem_g).wait()

    def remap_scatter(dst_v, rows_v):
      for j in range(CH // 16):
        d = dst_v[pl.ds(j * 16, 16)]
        t = d - base
        ok = (t >= 0) & (t < WIN)
        dst_v[pl.ds(j * 16, 16)] = jnp.where(ok, t + arow, DUMMY + arow)
      pltpu.sync_copy(rows_v, acc.at[dst_v], add=True)

    issue_idx(0, src_a, dst_a)
    wait_idx(src_a, dst_a)
    issue_gather(src_a, rows_a)
    issue_idx(1, src_b, dst_b)

    def body(p, carry):
      a = 2 * p
      wait_gather(src_a, rows_a)
      wait_idx(src_b, dst_b)
      issue_gather(src_b, rows_b)
      remap_scatter(dst_a, rows_a)
      issue_idx(a + 2, src_a, dst_a)
      wait_gather(src_b, rows_b)
      wait_idx(src_a, dst_a)
      issue_gather(src_a, rows_a)
      remap_scatter(dst_b, rows_b)
      issue_idx(a + 3, src_b, dst_b)
      return carry

    lax.fori_loop(0, npairs, body, 0)

    wait_gather(src_a, rows_a)
    wait_idx(src_b, dst_b)

    pltpu.sync_copy(acc.at[pl.ds(arow, WIN)],
                    out_hbm.at[cid].at[pl.ds(sid * WIN, WIN)])

  return sc_segment_sum


@functools.cache
def _make_sc_degree():
  mesh = plsc.VectorSubcoreMesh(core_axis_name="c", subcore_axis_name="s")
  cp = pltpu.CompilerParams()
  if "needs_layout_passes" in pltpu.CompilerParams.__dataclass_fields__:
    cp = dataclasses.replace(cp, needs_layout_passes=False)

  @functools.partial(
      pl.kernel,
      out_type=jax.ShapeDtypeStruct((NC, NS * WIN, H), jnp.float32),
      mesh=mesh,
      compiler_params=cp,
      scratch_types=[
          pltpu.VMEM((16,), jnp.int32),
          pltpu.VMEM((16,), jnp.int32),
          pltpu.VMEM((CH,), jnp.int32),
          pltpu.VMEM((CH,), jnp.int32),
          pltpu.VMEM((CH, H), jnp.float32),
          pltpu.VMEM_SHARED((NS * ACC_ROWS, H), jnp.float32),
          pltpu.SemaphoreType.DMA,
      ],
  )
  def sc_degree(dst_hbm, lo_hbm, nch_hbm, ones_hbm, zeros_hbm,
                out_hbm, lo_v, nch_v, dst_a, dst_b, ones_v, acc, sem_i):
    cid = lax.axis_index("c")
    sid = lax.axis_index("s")
    wid = cid * NS + sid
    base = wid * WIN
    arow = sid * ACC_ROWS

    pltpu.sync_copy(ones_hbm, ones_v)
    pltpu.sync_copy(zeros_hbm, acc.at[pl.ds(arow, CH)])
    pltpu.sync_copy(zeros_hbm, acc.at[pl.ds(arow + CH, CH)])
    pltpu.sync_copy(zeros_hbm.at[pl.ds(0, ACC_ROWS - 2 * CH)],
                    acc.at[pl.ds(arow + 2 * CH, ACC_ROWS - 2 * CH)])

    boff = pl.multiple_of(wid * 16, 16)
    pltpu.sync_copy(lo_hbm.at[pl.ds(boff, 16)], lo_v)
    pltpu.sync_copy(nch_hbm.at[pl.ds(boff, 16)], nch_v)
    lo = jnp.max(lo_v[...], axis=0)
    npairs = jnp.max(nch_v[...], axis=0)

    def issue_idx(c, dst_v):
      off = pl.multiple_of(lo + c * CH, CH)
      pltpu.async_copy(dst_hbm.at[pl.ds(off, CH)], dst_v, sem_i)

    def wait_idx(dst_v):
      pltpu.make_async_copy(dst_hbm.at[pl.ds(0, CH)], dst_v, sem_i).wait()

    def remap_scatter(dst_v):
      for j in range(CH // 16):
        d = dst_v[pl.ds(j * 16, 16)]
        t = d - base
        ok = (t >= 0) & (t < WIN)
        dst_v[pl.ds(j * 16, 16)] = jnp.where(ok, t + arow, DUMMY + arow)
      pltpu.sync_copy(ones_v, acc.at[dst_v], add=True)

    issue_idx(0, dst_a)
    wait_idx(dst_a)
    issue_idx(1, dst_b)

    def body(p, carry):
      a = 2 * p
      remap_scatter(dst_a)
      wait_idx(dst_b)
      issue_idx(a + 2, dst_a)
      remap_scatter(dst_b)
      wait_idx(dst_a)
      issue_idx(a + 3, dst_b)
      return carry

    lax.fori_loop(0, npairs, body, 0)

    wait_idx(dst_b)

    pltpu.sync_copy(acc.at[pl.ds(arow, WIN)],
                    out_hbm.at[cid].at[pl.ds(sid * WIN, WIN)])

  return sc_degree


_RB = 320
_NB = NPAD // _RB


def _combine_body(m_ref, c_ref, h_ref, wl_ref, wr_ref, b_ref, o_ref):
  cnt = c_ref[:, 0:1]
  inv = 1.0 / jnp.maximum(cnt, 1.0)
  agg = m_ref[...] * inv
  z = (jnp.dot(agg, wl_ref[...], preferred_element_type=jnp.float32)
       + jnp.dot(h_ref[...], wr_ref[...], preferred_element_type=jnp.float32)
       + b_ref[...])
  o_ref[...] = jnp.maximum(z, 0.0)


def _tc_combine(msg, cntp, h, wl, wr, b):
  return pl.pallas_call(
      _combine_body,
      grid=(_NB,),
      in_specs=[
          pl.BlockSpec((_RB, H), lambda i: (i, 0)),
          pl.BlockSpec((_RB, H), lambda i: (i, 0)),
          pl.BlockSpec((_RB, H), lambda i: (i, 0)),
          pl.BlockSpec((H, H), lambda i: (0, 0)),
          pl.BlockSpec((H, H), lambda i: (0, 0)),
          pl.BlockSpec((1, H), lambda i: (0, 0)),
      ],
      out_specs=pl.BlockSpec((_RB, H), lambda i: (i, 0)),
      out_shape=jax.ShapeDtypeStruct((NPAD, H), jnp.float32),
  )(msg, cntp, h, wl, wr, b.reshape(1, H))


def _pool_body(h_ref, b_ref, wo_ref, bo_ref, o_ref, acc_ref, cacc_ref):
  i = pl.program_id(0)

  @pl.when(i == 0)
  def _():
    acc_ref[...] = jnp.zeros_like(acc_ref)
    cacc_ref[...] = jnp.zeros_like(cacc_ref)

  h = h_ref[...]
  bidx = b_ref[...]
  iota_g = lax.broadcasted_iota(jnp.int32, (_RB, G), 1)
  onehot = (bidx == iota_g).astype(jnp.float32)
  acc_ref[...] += lax.dot_general(
      onehot, h, (((0,), (0,)), ((), ())), preferred_element_type=jnp.float32)
  cacc_ref[...] += lax.dot_general(
      onehot, jnp.ones((_RB, G), jnp.float32), (((0,), (0,)), ((), ())),
      preferred_element_type=jnp.float32)

  @pl.when(i == _NB - 1)
  def _():
    cnt = jnp.maximum(cacc_ref[:, 0:1], 1.0)
    pooled = acc_ref[...] / cnt
    logits = (jnp.dot(pooled, wo_ref[...], preferred_element_type=jnp.float32)
              + bo_ref[...])
    m = jnp.max(logits, axis=1, keepdims=True)
    lse = jnp.log(jnp.sum(jnp.exp(logits - m), axis=1, keepdims=True)) + m
    o_ref[...] = logits - lse


def _tc_pool(h, batch2d, wo, bo):
  return pl.pallas_call(
      _pool_body,
      grid=(_NB,),
      in_specs=[
          pl.BlockSpec((_RB, H), lambda i: (i, 0)),
          pl.BlockSpec((_RB, 1), lambda i: (i, 0)),
          pl.BlockSpec((H, C), lambda i: (0, 0)),
          pl.BlockSpec((1, C), lambda i: (0, 0)),
      ],
      out_specs=pl.BlockSpec((G, C), lambda i: (0, 0)),
      out_shape=jax.ShapeDtypeStruct((G, C), jnp.float32),
      scratch_shapes=[
          pltpu.VMEM((G, H), jnp.float32),
          pltpu.VMEM((G, G), jnp.float32),
      ],
  )(h, batch2d, wo, bo.reshape(1, C))


def kernel(x, edge_index, batch, Wl1, Wr1, b1, Wl2, Wr2, b2, Wl3, Wr3, b3,
           Wl4, Wr4, b4, Wl5, Wr5, b5, Wo, bo):
  src = edge_index[0].astype(jnp.int32)
  dst = edge_index[1].astype(jnp.int32)

  keys = jnp.sort(dst * (1 << KSHIFT) + src)
  keys = jnp.concatenate(
      [keys, jnp.full((EPS - E,), NPAD << KSHIFT, jnp.int32)])
  srcs = keys & ((1 << KSHIFT) - 1)
  dsts = keys >> KSHIFT

  bounds = jnp.searchsorted(dsts, jnp.arange(0, NPAD + 1, WIN)).astype(
      jnp.int32)
  lo = bounds[:-1]
  hi = bounds[1:]
  lo128 = (lo // CH) * CH
  nch = (hi - lo128 + (CH - 1)) // CH
  npairs = (nch + 1) // 2
  lo_b = jnp.repeat(lo128, 16)
  nch_b = jnp.repeat(nch, 16)
  npairs_b = jnp.repeat(npairs, 16)

  zeros_h = jnp.zeros((CH, H), jnp.float32)
  ones_h = jnp.ones((CH, H), jnp.float32)

  cntp = _make_sc_degree()(dsts, lo_b, npairs_b, ones_h,
                           zeros_h).reshape(NPAD, H)

  seg = _make_sc_segment_sum()

  h = jnp.concatenate([x, jnp.zeros((NPAD - N, H), jnp.float32)])
  for wl, wr, b in ((Wl1, Wr1, b1), (Wl2, Wr2, b2), (Wl3, Wr3, b3),
                    (Wl4, Wr4, b4), (Wl5, Wr5, b5)):
    msg = seg(h, srcs, dsts, lo_b, npairs_b, zeros_h).reshape(NPAD, H)
    h = _tc_combine(msg, cntp, h, wl, wr, b)

  batchp = jnp.concatenate([batch.astype(jnp.int32),
                            jnp.full((NPAD - N,), G, jnp.int32)])
  return _tc_pool(h, batchp.reshape(NPAD, 1), Wo, bo)


# speedup vs baseline: 2.2805x; 1.0010x over previous
import dataclasses
import functools

import jax
import jax.numpy as jnp
from jax import lax
from jax.experimental import pallas as pl
from jax.experimental.pallas import tpu as pltpu
from jax.experimental.pallas import tpu_sc as plsc

N = 10000
E = 320000
D = 128
H = 128
C = 64
G = 128

NC = 2
NS = 16
NW = NC * NS

CH = 128
WIN = 320
NPAD = NW * WIN
DUMMY = 328
ACC_ROWS = 336
EPS = 320512
KSHIFT = 14


@functools.cache
def _make_sc_segment_sum():
  mesh = plsc.VectorSubcoreMesh(core_axis_name="c", subcore_axis_name="s")
  cp = pltpu.CompilerParams()
  if "needs_layout_passes" in pltpu.CompilerParams.__dataclass_fields__:
    cp = dataclasses.replace(cp, needs_layout_passes=False)

  @functools.partial(
      pl.kernel,
      out_type=jax.ShapeDtypeStruct((NC, NS * WIN, H), jnp.float32),
      mesh=mesh,
      compiler_params=cp,
      scratch_types=[
          pltpu.VMEM((16,), jnp.int32),
          pltpu.VMEM((16,), jnp.int32),
          pltpu.VMEM((CH,), jnp.int32),
          pltpu.VMEM((CH,), jnp.int32),
          pltpu.VMEM((CH,), jnp.int32),
          pltpu.VMEM((CH,), jnp.int32),
          pltpu.VMEM((CH, H), jnp.float32),
          pltpu.VMEM((CH, H), jnp.float32),
          pltpu.VMEM_SHARED((NS * ACC_ROWS, H), jnp.float32),
          pltpu.SemaphoreType.DMA,
          pltpu.SemaphoreType.DMA,
      ],
  )
  def sc_segment_sum(h_hbm, src_hbm, dst_hbm, lo_hbm, nch_hbm, zeros_hbm,
                     out_hbm, lo_v, nch_v, src_a, dst_a, src_b, dst_b,
                     rows_a, rows_b, acc, sem_g, sem_i):
    cid = lax.axis_index("c")
    sid = lax.axis_index("s")
    wid = cid * NS + sid
    base = wid * WIN
    arow = sid * ACC_ROWS

    pltpu.sync_copy(zeros_hbm, acc.at[pl.ds(arow, CH)])
    pltpu.sync_copy(zeros_hbm, acc.at[pl.ds(arow + CH, CH)])
    pltpu.sync_copy(zeros_hbm.at[pl.ds(0, ACC_ROWS - 2 * CH)],
                    acc.at[pl.ds(arow + 2 * CH, ACC_ROWS - 2 * CH)])

    boff = pl.multiple_of(wid * 16, 16)
    pltpu.sync_copy(lo_hbm.at[pl.ds(boff, 16)], lo_v)
    pltpu.sync_copy(nch_hbm.at[pl.ds(boff, 16)], nch_v)
    lo = jnp.max(lo_v[...], axis=0)
    npairs = jnp.max(nch_v[...], axis=0)

    def issue_idx(c, src_v, dst_v):
      off = pl.multiple_of(lo + c * CH, CH)
      pltpu.async_copy(src_hbm.at[pl.ds(off, CH)], src_v, sem_i)
      pltpu.async_copy(dst_hbm.at[pl.ds(off, CH)], dst_v, sem_i)

    def wait_idx(src_v, dst_v):
      pltpu.make_async_copy(src_hbm.at[pl.ds(0, CH)], src_v, sem_i).wait()
      pltpu.make_async_copy(dst_hbm.at[pl.ds(0, CH)], dst_v, sem_i).wait()

    def issue_gather(src_v, rows_v):
      pltpu.async_copy(h_hbm.at[src_v], rows_v, sem_g)

    def wait_gather(src_v, rows_v):
      pltpu.make_async_copy(h_hbm.at[src_v], rows_v, sem_g).wait()

    def remap_scatter(dst_v, rows_v):
      for j in range(CH // 16):
        d = dst_v[pl.ds(j * 16, 16)]
        t = d - base
        ok = (t >= 0) & (t < WIN)
        dst_v[pl.ds(j * 16, 16)] = jnp.where(ok, t + arow, DUMMY + arow)
      pltpu.sync_copy(rows_v, acc.at[dst_v], add=True)

    issue_idx(0, src_a, dst_a)
    wait_idx(src_a, dst_a)
    issue_gather(src_a, rows_a)
    issue_idx(1, src_b, dst_b)

    def body(p, carry):
      a = 2 * p
      wait_gather(src_a, rows_a)
      wait_idx(src_b, dst_b)
      issue_gather(src_b, rows_b)
      remap_scatter(dst_a, rows_a)
      issue_idx(a + 2, src_a, dst_a)
      wait_gather(src_b, rows_b)
      wait_idx(src_a, dst_a)
      issue_gather(src_a, rows_a)
      remap_scatter(dst_b, rows_b)
      issue_idx(a + 3, src_b, dst_b)
      return carry

    lax.fori_loop(0, npairs, body, 0)

    wait_gather(src_a, rows_a)
    wait_idx(src_b, dst_b)

    pltpu.sync_copy(acc.at[pl.ds(arow, WIN)],
                    out_hbm.at[cid].at[pl.ds(sid * WIN, WIN)])

  return sc_segment_sum


@functools.cache
def _make_sc_degree():
  mesh = plsc.VectorSubcoreMesh(core_axis_name="c", subcore_axis_name="s")
  cp = pltpu.CompilerParams()
  if "needs_layout_passes" in pltpu.CompilerParams.__dataclass_fields__:
    cp = dataclasses.replace(cp, needs_layout_passes=False)

  @functools.partial(
      pl.kernel,
      out_type=jax.ShapeDtypeStruct((NC, NS * WIN, H), jnp.float32),
      mesh=mesh,
      compiler_params=cp,
      scratch_types=[
          pltpu.VMEM((16,), jnp.int32),
          pltpu.VMEM((16,), jnp.int32),
          pltpu.VMEM((CH,), jnp.int32),
          pltpu.VMEM((CH,), jnp.int32),
          pltpu.VMEM((CH, H), jnp.float32),
          pltpu.VMEM_SHARED((NS * ACC_ROWS, H), jnp.float32),
          pltpu.SemaphoreType.DMA,
      ],
  )
  def sc_degree(dst_hbm, lo_hbm, nch_hbm, ones_hbm, zeros_hbm,
                out_hbm, lo_v, nch_v, dst_a, dst_b, ones_v, acc, sem_i):
    cid = lax.axis_index("c")
    sid = lax.axis_index("s")
    wid = cid * NS + sid
    base = wid * WIN
    arow = sid * ACC_ROWS

    pltpu.sync_copy(ones_hbm, ones_v)
    pltpu.sync_copy(zeros_hbm, acc.at[pl.ds(arow, CH)])
    pltpu.sync_copy(zeros_hbm, acc.at[pl.ds(arow + CH, CH)])
    pltpu.sync_copy(zeros_hbm.at[pl.ds(0, ACC_ROWS - 2 * CH)],
                    acc.at[pl.ds(arow + 2 * CH, ACC_ROWS - 2 * CH)])

    boff = pl.multiple_of(wid * 16, 16)
    pltpu.sync_copy(lo_hbm.at[pl.ds(boff, 16)], lo_v)
    pltpu.sync_copy(nch_hbm.at[pl.ds(boff, 16)], nch_v)
    lo = jnp.max(lo_v[...], axis=0)
    npairs = jnp.max(nch_v[...], axis=0)

    def issue_idx(c, dst_v):
      off = pl.multiple_of(lo + c * CH, CH)
      pltpu.async_copy(dst_hbm.at[pl.ds(off, CH)], dst_v, sem_i)

    def wait_idx(dst_v):
      pltpu.make_async_copy(dst_hbm.at[pl.ds(0, CH)], dst_v, sem_i).wait()

    def remap_scatter(dst_v):
      for j in range(CH // 16):
        d = dst_v[pl.ds(j * 16, 16)]
        t = d - base
        ok = (t >= 0) & (t < WIN)
        dst_v[pl.ds(j * 16, 16)] = jnp.where(ok, t + arow, DUMMY + arow)
      pltpu.sync_copy(ones_v, acc.at[dst_v], add=True)

    issue_idx(0, dst_a)
    wait_idx(dst_a)
    issue_idx(1, dst_b)

    def body(p, carry):
      a = 2 * p
      remap_scatter(dst_a)
      wait_idx(dst_b)
      issue_idx(a + 2, dst_a)
      remap_scatter(dst_b)
      wait_idx(dst_a)
      issue_idx(a + 3, dst_b)
      return carry

    lax.fori_loop(0, npairs, body, 0)

    wait_idx(dst_b)

    pltpu.sync_copy(acc.at[pl.ds(arow, WIN)],
                    out_hbm.at[cid].at[pl.ds(sid * WIN, WIN)])

  return sc_degree


_RB = 320
_NB = NPAD // _RB


def _combine_body(m_ref, c_ref, h_ref, wl_ref, wr_ref, b_ref, o_ref):
  cnt = c_ref[:, 0:1]
  inv = 1.0 / jnp.maximum(cnt, 1.0)
  agg = m_ref[...] * inv
  z = (jnp.dot(agg, wl_ref[...], preferred_element_type=jnp.float32)
       + jnp.dot(h_ref[...], wr_ref[...], preferred_element_type=jnp.float32)
       + b_ref[...])
  o_ref[...] = jnp.maximum(z, 0.0)


def _tc_combine(msg, cntp, h, wl, wr, b):
  return pl.pallas_call(
      _combine_body,
      grid=(_NB,),
      in_specs=[
          pl.BlockSpec((_RB, H), lambda i: (i, 0)),
          pl.BlockSpec((_RB, H), lambda i: (i, 0)),
          pl.BlockSpec((_RB, H), lambda i: (i, 0)),
          pl.BlockSpec((H, H), lambda i: (0, 0)),
          pl.BlockSpec((H, H), lambda i: (0, 0)),
          pl.BlockSpec((1, H), lambda i: (0, 0)),
      ],
      out_specs=pl.BlockSpec((_RB, H), lambda i: (i, 0)),
      out_shape=jax.ShapeDtypeStruct((NPAD, H), jnp.float32),
  )(msg, cntp, h, wl, wr, b.reshape(1, H))


def _pool_body(h_ref, b_ref, wo_ref, bo_ref, o_ref, acc_ref, cacc_ref):
  i = pl.program_id(0)

  @pl.when(i == 0)
  def _():
    acc_ref[...] = jnp.zeros_like(acc_ref)
    cacc_ref[...] = jnp.zeros_like(cacc_ref)

  h = h_ref[...]
  bidx = b_ref[...]
  iota_g = lax.broadcasted_iota(jnp.int32, (_RB, G), 1)
  onehot = (bidx == iota_g).astype(jnp.float32)
  acc_ref[...] += lax.dot_general(
      onehot, h, (((0,), (0,)), ((), ())), preferred_element_type=jnp.float32)
  cacc_ref[...] += lax.dot_general(
      onehot, jnp.ones((_RB, G), jnp.float32), (((0,), (0,)), ((), ())),
      preferred_element_type=jnp.float32)

  @pl.when(i == _NB - 1)
  def _():
    cnt = jnp.maximum(cacc_ref[:, 0:1], 1.0)
    pooled = acc_ref[...] / cnt
    logits = (jnp.dot(pooled, wo_ref[...], preferred_element_type=jnp.float32)
              + bo_ref[...])
    m = jnp.max(logits, axis=1, keepdims=True)
    lse = jnp.log(jnp.sum(jnp.exp(logits - m), axis=1, keepdims=True)) + m
    o_ref[...] = logits - lse


def _tc_pool(h, batch2d, wo, bo):
  return pl.pallas_call(
      _pool_body,
      grid=(_NB,),
      in_specs=[
          pl.BlockSpec((_RB, H), lambda i: (i, 0)),
          pl.BlockSpec((_RB, 1), lambda i: (i, 0)),
          pl.BlockSpec((H, C), lambda i: (0, 0)),
          pl.BlockSpec((1, C), lambda i: (0, 0)),
      ],
      out_specs=pl.BlockSpec((G, C), lambda i: (0, 0)),
      out_shape=jax.ShapeDtypeStruct((G, C), jnp.float32),
      scratch_shapes=[
          pltpu.VMEM((G, H), jnp.float32),
          pltpu.VMEM((G, G), jnp.float32),
      ],
  )(h, batch2d, wo, bo.reshape(1, C))


def kernel(x, edge_index, batch, Wl1, Wr1, b1, Wl2, Wr2, b2, Wl3, Wr3, b3,
           Wl4, Wr4, b4, Wl5, Wr5, b5, Wo, bo):
  src = edge_index[0].astype(jnp.int32)
  dst = edge_index[1].astype(jnp.int32)

  keys = jnp.sort(dst * (1 << KSHIFT) + src)
  keys = jnp.concatenate(
      [keys, jnp.full((EPS - E,), NPAD << KSHIFT, jnp.int32)])
  srcs = keys & ((1 << KSHIFT) - 1)
  dsts = keys >> KSHIFT

  bounds = jnp.searchsorted(dsts, jnp.arange(0, NPAD + 1, WIN)).astype(
      jnp.int32)
  lo = bounds[:-1]
  hi = bounds[1:]
  lo128 = (lo // CH) * CH
  nch = (hi - lo128 + (CH - 1)) // CH
  npairs = (nch + 1) // 2
  lo_b = jnp.repeat(lo128, 16)
  npairs_b = jnp.repeat(npairs, 16)

  zeros_h = jnp.zeros((CH, H), jnp.float32)
  ones_h = jnp.ones((CH, H), jnp.float32)

  cntp = _make_sc_degree()(dsts, lo_b, npairs_b, ones_h,
                           zeros_h).reshape(NPAD, H)

  seg = _make_sc_segment_sum()

  h = jnp.concatenate([x, jnp.zeros((NPAD - N, H), jnp.float32)])
  for wl, wr, b in ((Wl1, Wr1, b1), (Wl2, Wr2, b2), (Wl3, Wr3, b3),
                    (Wl4, Wr4, b4), (Wl5, Wr5, b5)):
    msg = seg(h, srcs, dsts, lo_b, npairs_b, zeros_h).reshape(NPAD, H)
    h = _tc_combine(msg, cntp, h, wl, wr, b)

  batchp = jnp.concatenate([batch.astype(jnp.int32),
                            jnp.full((NPAD - N,), G, jnp.int32)])
  return _tc_pool(h, batchp.reshape(NPAD, 1), Wo, bo)
